# baseline (speedup 1.0000x reference)
import jax
import jax.numpy as jnp
from jax import lax
from jax.experimental import pallas as pl
from jax.experimental.pallas import tpu as pltpu
from jax.experimental.pallas import tpu_sc as plsc

NG = 2000
NN = 10000
DD = 128
NQ = 4
DQ = DD // NQ
EE = 320000
NC = 2
NS = 16
EPT = EE // NS
CH = 80
NCHUNK = EPT // CH
SROWS = 624
ZC = 48
TAIL = NN - NS * SROWS
U = 5
NB = NCHUNK // U
PBITS = 14

_mesh = plsc.VectorSubcoreMesh(
    core_axis_name="c", subcore_axis_name="s", num_cores=NC, num_subcores=NS)


def _make_sc_layer():
    def body(x_hbm, cr_hbm, w_hbm, out_hbm,
             cr_v, w_v, col_ring, row_ring,
             b0, b1, b2, b3, b4, stage, y_sh, x_sh,
             g0, g1, g2, g3, g4, s0, s1, s2, s3, s4):
        bufs = (b0, b1, b2, b3, b4)
        gsems = (g0, g1, g2, g3, g4)
        ssems = (s0, s1, s2, s3, s4)
        c = lax.axis_index("c")
        s = lax.axis_index("s")
        row0 = s * SROWS

        for e in range(ZC):
            for k in range(DQ // 16):
                stage[e, pl.ds(k * 16, 16)] = jnp.zeros((16,), jnp.float32)

        pltpu.sync_copy(cr_hbm.at[pl.ds(s * EPT, EPT)], cr_v)
        pltpu.sync_copy(w_hbm.at[pl.ds(s * EPT, EPT)], w_v)

        def unpack_idx(j, k):
            for g in range(CH // 16):
                cr16 = cr_v[pl.ds(j * CH + g * 16, 16)]
                col_ring[k, pl.ds(g * 16, 16)] = (
                    jnp.bitwise_and(cr16, (1 << PBITS) - 1))
                row_ring[k, pl.ds(g * 16, 16)] = (
                    lax.shift_right_logical(cr16, PBITS))

        def g_start(buf, k):
            pltpu.async_copy(x_sh.at[col_ring.at[k]], buf, gsems[k])

        def g_wait(buf, k):
            pltpu.make_async_copy(x_sh.at[col_ring.at[k]], buf,
                                  gsems[k]).wait()

        def s_start(buf, k):
            pltpu.async_copy(buf, y_sh.at[row_ring.at[k]], ssems[k], add=True)

        def s_wait(buf, k):
            pltpu.make_async_copy(buf, y_sh.at[row_ring.at[k]],
                                  ssems[k]).wait()

        def scale(j, buf):
            for g in range(CH // 16):
                w16 = w_v[pl.ds(j * CH + g * 16, 16)]
                for e in range(16):
                    wsplat = jnp.broadcast_to(w16[e], (16,))
                    for k in range(DQ // 16):
                        buf[g * 16 + e, pl.ds(k * 16, 16)] = (
                            buf[g * 16 + e, pl.ds(k * 16, 16)] * wsplat)

        @pl.loop(0, 2)
        def _pass(q):
            qq = c * 2 + q

            @pl.loop(0, SROWS // ZC)
            def _zero(i):
                pltpu.sync_copy(stage.at[pl.ds(0, ZC)],
                                y_sh.at[pl.ds(row0 + i * ZC, ZC)])

            pltpu.sync_copy(x_hbm.at[qq, pl.ds(row0, SROWS), :],
                            x_sh.at[pl.ds(row0, SROWS)])

            @pl.when(s == NS - 1)
            def _tails_in():
                pltpu.sync_copy(stage.at[pl.ds(0, TAIL)],
                                y_sh.at[pl.ds(NS * SROWS, TAIL)])
                pltpu.sync_copy(x_hbm.at[qq, pl.ds(NS * SROWS, TAIL), :],
                                x_sh.at[pl.ds(NS * SROWS, TAIL)])

            plsc.subcore_barrier()

            for k in range(U):
                unpack_idx(k, k)
                g_start(bufs[k], k)

            @pl.loop(0, NB - 1)
            def _chunk(i):
                j0 = i * U
                for k in range(U):
                    g_wait(bufs[k], k)
                    scale(j0 + k, bufs[k])
                    s_start(bufs[k], k)
                for k in range(U):
                    s_wait(bufs[k], k)
                    unpack_idx(j0 + U + k, k)
                    g_start(bufs[k], k)

            jlast = (NB - 1) * U
            for k in range(U):
                g_wait(bufs[k], k)
                scale(jlast + k, bufs[k])
                s_start(bufs[k], k)
            for k in range(U):
                s_wait(bufs[k], k)

            plsc.subcore_barrier()

            @pl.loop(0, SROWS // ZC)
            def _out(i):
                r = row0 + i * ZC
                pltpu.sync_copy(y_sh.at[pl.ds(r, ZC)],
                                stage.at[pl.ds(ZC, ZC)])
                pltpu.sync_copy(stage.at[pl.ds(ZC, ZC)],
                                out_hbm.at[qq, pl.ds(r, ZC), :])

            @pl.when(s == NS - 1)
            def _out_tail():
                pltpu.sync_copy(y_sh.at[pl.ds(NS * SROWS, TAIL)],
                                stage.at[pl.ds(ZC, TAIL)])
                pltpu.sync_copy(stage.at[pl.ds(ZC, TAIL)],
                                out_hbm.at[qq, pl.ds(NS * SROWS, TAIL), :])

            plsc.subcore_barrier()

    return pl.kernel(
        body,
        out_type=jax.ShapeDtypeStruct((NQ, NN, DQ), jnp.float32),
        mesh=_mesh,
        scratch_types=[
            pltpu.VMEM((EPT,), jnp.int32),
            pltpu.VMEM((EPT,), jnp.float32),
            pltpu.VMEM((U, CH), jnp.int32),
            pltpu.VMEM((U, CH), jnp.int32),
            pltpu.VMEM((CH, DQ), jnp.float32),
            pltpu.VMEM((CH, DQ), jnp.float32),
            pltpu.VMEM((CH, DQ), jnp.float32),
            pltpu.VMEM((CH, DQ), jnp.float32),
            pltpu.VMEM((CH, DQ), jnp.float32),
            pltpu.VMEM((2 * ZC, DQ), jnp.float32),
            pltpu.VMEM_SHARED((NN, DQ), jnp.float32),
            pltpu.VMEM_SHARED((NN, DQ), jnp.float32),
            pltpu.SemaphoreType.DMA,
            pltpu.SemaphoreType.DMA,
            pltpu.SemaphoreType.DMA,
            pltpu.SemaphoreType.DMA,
            pltpu.SemaphoreType.DMA,
            pltpu.SemaphoreType.DMA,
            pltpu.SemaphoreType.DMA,
            pltpu.SemaphoreType.DMA,
            pltpu.SemaphoreType.DMA,
            pltpu.SemaphoreType.DMA,
        ],
        compiler_params=pltpu.CompilerParams(use_tc_tiling_on_sc=False),
    )


_sc_layer = _make_sc_layer()


def _final(x0, y1, y2, y3):
    br = 400

    def body(a_ref, b_ref, c_ref, d_ref, o_ref):
        o_ref[0] = (a_ref[0] + b_ref[0] + c_ref[0] + d_ref[0]) * 0.25

    spec = pl.BlockSpec((1, br, DQ), lambda i, h: (h, i, 0))
    quarters = pl.pallas_call(
        body,
        out_shape=jax.ShapeDtypeStruct((NQ, NG, DQ), jnp.float32),
        grid=(NG // br, NQ),
        in_specs=[spec, spec, spec, spec],
        out_specs=pl.BlockSpec((1, br, DQ), lambda i, h: (h, i, 0)),
    )(x0, y1, y2, y3)
    return jnp.concatenate(
        [quarters[0], quarters[1], quarters[2], quarters[3]], axis=1)


def kernel(groups_emb, items_emb, edge_index, edge_weight):
    all_emb = jnp.concatenate([groups_emb, items_emb], axis=0)
    x0 = all_emb.reshape(NN, NQ, DQ).transpose(1, 0, 2)
    cr = (edge_index[0] << PBITS) | edge_index[1]

    y1 = _sc_layer(x0, cr, edge_weight)
    y2 = _sc_layer(y1, cr, edge_weight)
    y3 = _sc_layer(y2, cr, edge_weight)
    return _final(x0, y1, y2, y3)
